# baseline (speedup 1.0000x reference)
import functools

import jax
import jax.numpy as jnp
from jax import lax
from jax.experimental import pallas as pl
from jax.experimental.pallas import tpu as pltpu
from jax.experimental.pallas import tpu_sc as plsc

F = 26
V = 100000
D = 32
B = 4096
DENSE_IN = 13
LR = 512
NL = 3
CROSS_IN = (F + 1) * D
S_DIM = F * D

_NC = 2
_NS = 16
_NW = _NC * _NS
_GROUPS = B // 16


_NQ = 4
_QS = 24960
_QL = 25120
_DEPTH = 3


def _sc_gather_body(idx_hbm, table_hbm, out_hbm,
                    buf0, buf1, buf2, buf3, idxa, idxb, resa, resb,
                    rs0, rs1, rs2, rs3, is0, is1, os0, os1):
    w = lax.axis_index("s") * _NC + lax.axis_index("c")
    bufs, rsem = (buf0, buf1, buf2, buf3), (rs0, rs1, rs2, rs3)
    idxv, isem = (idxa, idxb), (is0, is1)
    resv, osem = (resa, resb), (os0, os1)
    n_items = _NQ * F

    _SUB = 12416

    def start_row(k):
        f, q = divmod(k, _NQ)
        if q == _NQ - 1:
            return [pltpu.async_copy(
                table_hbm.at[f, w, pl.ds(q * _QS, _QL)],
                bufs[k % _NQ].at[pl.ds(0, _QL)], rsem[k % _NQ])]
        return [
            pltpu.async_copy(
                table_hbm.at[f, w, pl.ds(q * _QS, _SUB)],
                bufs[k % _NQ].at[pl.ds(0, _SUB)], rsem[k % _NQ]),
            pltpu.async_copy(
                table_hbm.at[f, w, pl.ds(q * _QS + _SUB, _QS - _SUB)],
                bufs[k % _NQ].at[pl.ds(_SUB, _QS - _SUB)], rsem[k % _NQ]),
        ]

    def start_idx(f):
        return pltpu.async_copy(idx_hbm.at[f], idxv[f % 2], isem[f % 2])

    pend_row = {k: start_row(k) for k in range(_DEPTH)}
    pend_idx = {0: start_idx(0)}
    pend_out = {}
    lanes = lax.broadcasted_iota(jnp.int32, (16,), 0)
    for k in range(n_items):
        f, q = divmod(k, _NQ)
        if k + _DEPTH < n_items:
            pend_row[k + _DEPTH] = start_row(k + _DEPTH)
        if q == 0 and f + 1 < F:
            pend_idx[f + 1] = start_idx(f + 1)
        for _c in pend_row.pop(k):
            _c.wait()
        if q == 0:
            pend_idx.pop(f).wait()
            if f >= 2:
                pend_out.pop(f - 2).wait()
        buf, iv, rv = bufs[k % _NQ], idxv[f % 2], resv[f % 2]
        lo = q * _QS

        @plsc.parallel_loop(0, B, step=16, unroll=4)
        def body(g, iv=iv, rv=rv, buf=buf, lanes=lanes, lo=lo, q=q):
            idx16 = iv[pl.ds(g, 16)]
            if q == 0:
                m = idx16 < _QS
            elif q == _NQ - 1:
                m = idx16 >= lo
            else:
                m = (idx16 >= lo) & (idx16 < lo + _QS)
            gv = plsc.load_gather(buf, [idx16 - lo], mask=m)
            plsc.store_scatter(rv, [lanes + g], gv, mask=m)

        if q == _NQ - 1:
            pend_out[f] = pltpu.async_copy(rv, out_hbm.at[f * D + w],
                                           osem[f % 2])
    pend_out.pop(F - 2).wait()
    pend_out.pop(F - 1).wait()


@functools.cache
def _sc_gather():
    return pl.kernel(
        _sc_gather_body,
        out_type=jax.ShapeDtypeStruct((S_DIM, B), jnp.float32),
        mesh=plsc.VectorSubcoreMesh(core_axis_name="c", subcore_axis_name="s"),
        scratch_types=(
            [pltpu.VMEM((_QL,), jnp.float32)] * 4
            + [pltpu.VMEM((B,), jnp.int32)] * 2
            + [pltpu.VMEM((B,), jnp.float32)] * 2
            + [pltpu.SemaphoreType.DMA] * 8
        ),
        compiler_params=pltpu.CompilerParams(needs_layout_passes=False),
    )


def _mmT(w, x):
    return lax.dot_general(
        w.astype(jnp.bfloat16),
        x.astype(jnp.bfloat16),
        (((1,), (0,)), ((), ())),
        preferred_element_type=jnp.float32,
    )


def _mlp_body(x_ref, dw1, db1, dw2, db2, dw3, db3, out_ref):
    zero = jnp.float32(0.0)
    x = x_ref[...]
    h = jnp.maximum(_mmT(dw1[...], x) + db1[...], zero)
    h = jnp.maximum(_mmT(dw2[...], h) + db2[...], zero)
    out_ref[...] = jnp.maximum(_mmT(dw3[...], h) + db3[...], zero)


_MBB = 2048

_mlp_call = pl.pallas_call(
    _mlp_body,
    grid=(B // _MBB,),
    in_specs=[
        pl.BlockSpec((DENSE_IN, _MBB), lambda i: (0, i)),
        pl.BlockSpec((512, DENSE_IN), lambda i: (0, 0)),
        pl.BlockSpec((512, 1), lambda i: (0, 0)),
        pl.BlockSpec((256, 512), lambda i: (0, 0)),
        pl.BlockSpec((256, 1), lambda i: (0, 0)),
        pl.BlockSpec((D, 256), lambda i: (0, 0)),
        pl.BlockSpec((D, 1), lambda i: (0, 0)),
    ],
    out_specs=pl.BlockSpec((D, _MBB), lambda i: (0, i)),
    out_shape=jax.ShapeDtypeStruct((D, B), jnp.float32),
)


def _dense_body(d_ref, s_ref,
                vd, vs, wd, ws, bd, bs,
                o1d, o1s, ob1, ow2, ob2, ow3t, ob3, out_ref):
    zero = jnp.float32(0.0)
    d = d_ref[...]
    s = s_ref[...]
    xld, xls = d, s
    for l in range(NL):
        xv = _mmT(vd[l], xld) + _mmT(vs[l], xls)
        xld = d * (_mmT(wd[l], xv) + bd[l]) + xld
        xls = s * (_mmT(ws[l], xv) + bs[l]) + xls
    h = jnp.maximum(_mmT(o1d[...], xld) + _mmT(o1s[...], xls) + ob1[...],
                    zero)
    h = jnp.maximum(_mmT(ow2[...], h) + ob2[...], zero)
    out_ref[...] = jnp.sum(h * ow3t[...], axis=0, keepdims=True) + ob3[...]


_BB = 2048
_GRID = B // _BB


def _full(shape):
    return pl.BlockSpec(shape, lambda i: (0,) * len(shape))


_dense_call = pl.pallas_call(
    _dense_body,
    grid=(_GRID,),
    in_specs=[
        pl.BlockSpec((D, _BB), lambda i: (0, i)),
        pl.BlockSpec((S_DIM, _BB), lambda i: (0, i)),
        _full((NL, LR, D)), _full((NL, LR, S_DIM)),
        _full((NL, D, LR)), _full((NL, S_DIM, LR)),
        _full((NL, D, 1)), _full((NL, S_DIM, 1)),
        _full((512, D)), _full((512, S_DIM)), _full((512, 1)),
        _full((256, 512)), _full((256, 1)),
        _full((256, 1)), _full((1, 1)),
    ],
    out_specs=pl.BlockSpec((1, _BB), lambda i: (0, i)),
    out_shape=jax.ShapeDtypeStruct((1, B), jnp.float32),
)


def kernel(dense_features, sparse_indices, tables, dw1, db1, dw2, db2, dw3,
           db3, cnV, cnW, cnB, ow1, ob1, ow2, ob2, ow3, ob3):
    idx_t = sparse_indices.T
    tables_t = tables.transpose(0, 2, 1)
    s_t = _sc_gather()(idx_t, tables_t)

    vd, vs = cnV[:, :, :D], cnV[:, :, D:]
    wd, ws = cnW[:, :D, :], cnW[:, D:, :]
    bd, bs = cnB[:, :D, None], cnB[:, D:, None]
    o1d, o1s = ow1[:, :D], ow1[:, D:]

    d_t = _mlp_call(dense_features.T, dw1, db1[:, None], dw2, db2[:, None],
                    dw3, db3[:, None])
    logits_t = _dense_call(
        d_t, s_t,
        vd, vs, wd, ws, bd, bs,
        o1d, o1s, ob1[:, None], ow2, ob2[:, None], ow3.T, ob3[:, None],
    )
    return logits_t.reshape(B, 1)


# speedup vs baseline: 1.0041x; 1.0041x over previous
import functools

import jax
import jax.numpy as jnp
from jax import lax
from jax.experimental import pallas as pl
from jax.experimental.pallas import tpu as pltpu
from jax.experimental.pallas import tpu_sc as plsc

F = 26
V = 100000
D = 32
B = 4096
DENSE_IN = 13
LR = 512
NL = 3
CROSS_IN = (F + 1) * D
S_DIM = F * D

_NC = 2
_NQ = 4
_QS = 24960
_QL = 25120
_DEPTH = 3


def _sc_gather_body(idx_hbm, table_hbm, out_hbm,
                    buf0, buf1, buf2, buf3, idxa, idxb, resa, resb,
                    rs0, rs1, rs2, rs3, is0, is1, os0, os1):
    w = lax.axis_index("s") * _NC + lax.axis_index("c")
    bufs, rsem = (buf0, buf1, buf2, buf3), (rs0, rs1, rs2, rs3)
    idxv, isem = (idxa, idxb), (is0, is1)
    resv, osem = (resa, resb), (os0, os1)
    n_items = _NQ * F

    _SUB = 12416

    def start_row(k):
        f, q = divmod(k, _NQ)
        if q == _NQ - 1:
            return [pltpu.async_copy(
                table_hbm.at[f, w, pl.ds(q * _QS, _QL)],
                bufs[k % _NQ].at[pl.ds(0, _QL)], rsem[k % _NQ])]
        return [
            pltpu.async_copy(
                table_hbm.at[f, w, pl.ds(q * _QS, _SUB)],
                bufs[k % _NQ].at[pl.ds(0, _SUB)], rsem[k % _NQ]),
            pltpu.async_copy(
                table_hbm.at[f, w, pl.ds(q * _QS + _SUB, _QS - _SUB)],
                bufs[k % _NQ].at[pl.ds(_SUB, _QS - _SUB)], rsem[k % _NQ]),
        ]

    def start_idx(f):
        return pltpu.async_copy(idx_hbm.at[f], idxv[f % 2], isem[f % 2])

    pend_row = {k: start_row(k) for k in range(_DEPTH)}
    pend_idx = {0: start_idx(0)}
    pend_out = {}
    lanes = lax.broadcasted_iota(jnp.int32, (16,), 0)
    for k in range(n_items):
        f, q = divmod(k, _NQ)
        if k + _DEPTH < n_items:
            pend_row[k + _DEPTH] = start_row(k + _DEPTH)
        if q == 0 and f + 1 < F:
            pend_idx[f + 1] = start_idx(f + 1)
        for _c in pend_row.pop(k):
            _c.wait()
        if q == 0:
            pend_idx.pop(f).wait()
            if f >= 2:
                pend_out.pop(f - 2).wait()
        buf, iv, rv = bufs[k % _NQ], idxv[f % 2], resv[f % 2]
        lo = q * _QS

        @plsc.parallel_loop(0, B, step=16, unroll=4)
        def body(g, iv=iv, rv=rv, buf=buf, lanes=lanes, lo=lo, q=q):
            idx16 = iv[pl.ds(g, 16)]
            if q == 0:
                m = idx16 < _QS
            elif q == _NQ - 1:
                m = idx16 >= lo
            else:
                m = (idx16 >= lo) & (idx16 < lo + _QS)
            gv = plsc.load_gather(buf, [idx16 - lo], mask=m)
            plsc.store_scatter(rv, [lanes + g], gv, mask=m)

        if q == _NQ - 1:
            pend_out[f] = pltpu.async_copy(rv, out_hbm.at[f * D + w],
                                           osem[f % 2])
    pend_out.pop(F - 2).wait()
    pend_out.pop(F - 1).wait()


@functools.cache
def _sc_gather():
    return pl.kernel(
        _sc_gather_body,
        out_type=jax.ShapeDtypeStruct((S_DIM, B), jnp.float32),
        mesh=plsc.VectorSubcoreMesh(core_axis_name="c", subcore_axis_name="s"),
        scratch_types=(
            [pltpu.VMEM((_QL,), jnp.float32)] * 4
            + [pltpu.VMEM((B,), jnp.int32)] * 2
            + [pltpu.VMEM((B,), jnp.float32)] * 2
            + [pltpu.SemaphoreType.DMA] * 8
        ),
        compiler_params=pltpu.CompilerParams(needs_layout_passes=False),
    )


def _mmT(w, x):
    return lax.dot_general(
        w.astype(jnp.bfloat16),
        x.astype(jnp.bfloat16),
        (((1,), (0,)), ((), ())),
        preferred_element_type=jnp.float32,
    )


def _mlp_body(x_ref, dw1, db1, dw2, db2, dw3, db3, out_ref):
    zero = jnp.float32(0.0)
    x = x_ref[...]
    h = jnp.maximum(_mmT(dw1[...], x) + db1[...], zero)
    h = jnp.maximum(_mmT(dw2[...], h) + db2[...], zero)
    out_ref[...] = jnp.maximum(_mmT(dw3[...], h) + db3[...], zero)


_MBB = 2048

_mlp_call = pl.pallas_call(
    _mlp_body,
    grid=(B // _MBB,),
    in_specs=[
        pl.BlockSpec((DENSE_IN, _MBB), lambda i: (0, i)),
        pl.BlockSpec((512, DENSE_IN), lambda i: (0, 0)),
        pl.BlockSpec((512, 1), lambda i: (0, 0)),
        pl.BlockSpec((256, 512), lambda i: (0, 0)),
        pl.BlockSpec((256, 1), lambda i: (0, 0)),
        pl.BlockSpec((D, 256), lambda i: (0, 0)),
        pl.BlockSpec((D, 1), lambda i: (0, 0)),
    ],
    out_specs=pl.BlockSpec((D, _MBB), lambda i: (0, i)),
    out_shape=jax.ShapeDtypeStruct((D, B), jnp.float32),
)


def _dense_body(d_ref, s_ref,
                vd, vs, wd, ws, bd, bs,
                o1d, o1s, ob1, ow2, ob2, ow3t, ob3, out_ref):
    zero = jnp.float32(0.0)
    d = d_ref[...]
    s = s_ref[...]
    xld, xls = d, s
    for l in range(NL):
        xv = _mmT(vd[l], xld) + _mmT(vs[l], xls)
        xld = d * (_mmT(wd[l], xv) + bd[l]) + xld
        xls = s * (_mmT(ws[l], xv) + bs[l]) + xls
    h = jnp.maximum(_mmT(o1d[...], xld) + _mmT(o1s[...], xls) + ob1[...],
                    zero)
    h = jnp.maximum(_mmT(ow2[...], h) + ob2[...], zero)
    out_ref[...] = jnp.sum(h * ow3t[...], axis=0, keepdims=True) + ob3[...]


_BB = 2048
_GRID = B // _BB


def _full(shape):
    return pl.BlockSpec(shape, lambda i: (0,) * len(shape))


_dense_call = pl.pallas_call(
    _dense_body,
    grid=(_GRID,),
    in_specs=[
        pl.BlockSpec((D, _BB), lambda i: (0, i)),
        pl.BlockSpec((S_DIM, _BB), lambda i: (0, i)),
        _full((NL, LR, D)), _full((NL, LR, S_DIM)),
        _full((NL, D, LR)), _full((NL, S_DIM, LR)),
        _full((NL, D, 1)), _full((NL, S_DIM, 1)),
        _full((512, D)), _full((512, S_DIM)), _full((512, 1)),
        _full((256, 512)), _full((256, 1)),
        _full((256, 1)), _full((1, 1)),
    ],
    out_specs=pl.BlockSpec((1, _BB), lambda i: (0, i)),
    out_shape=jax.ShapeDtypeStruct((1, B), jnp.float32),
)


def kernel(dense_features, sparse_indices, tables, dw1, db1, dw2, db2, dw3,
           db3, cnV, cnW, cnB, ow1, ob1, ow2, ob2, ow3, ob3):
    idx_t = sparse_indices.T
    tables_t = tables.transpose(0, 2, 1)
    s_t = _sc_gather()(idx_t, tables_t)

    vd, vs = cnV[:, :, :D], cnV[:, :, D:]
    wd, ws = cnW[:, :D, :], cnW[:, D:, :]
    bd, bs = cnB[:, :D, None], cnB[:, D:, None]
    o1d, o1s = ow1[:, :D], ow1[:, D:]

    d_t = _mlp_call(dense_features.T, dw1, db1[:, None], dw2, db2[:, None],
                    dw3, db3[:, None])
    logits_t = _dense_call(
        d_t, s_t,
        vd, vs, wd, ws, bd, bs,
        o1d, o1s, ob1[:, None], ow2, ob2[:, None], ow3.T, ob3[:, None],
    )
    return logits_t.reshape(B, 1)
